# baseline (speedup 1.0000x reference)
import jax
import jax.numpy as jnp
from jax.experimental import pallas as pl
from jax.experimental.pallas import tpu as pltpu

N = 16384
D = 256
K = 1024
BN = 2048


def _assign_kernel(x_ref, c_ref, out_ref, cn_ref):
    @pl.when(pl.program_id(0) == 0)
    def _():
        c = c_ref[...]
        cn = jnp.sum(c * c, axis=0, keepdims=True)
        cn_ref[...] = cn.reshape(K, 1)

    prod_t = jax.lax.dot_general(
        c_ref[...], x_ref[...],
        dimension_numbers=(((0,), (1,)), ((), ())),
        preferred_element_type=jnp.float32)
    scores = cn_ref[...] - 2.0 * prod_t
    am = jnp.argmin(scores, axis=0).astype(jnp.int32)
    out_ref[...] = am.reshape(1, 1, BN)


def kernel(x, centroids):
    out = pl.pallas_call(
        _assign_kernel,
        grid=(N // BN,),
        in_specs=[
            pl.BlockSpec((BN, D), lambda i: (i, 0)),
            pl.BlockSpec((D, K), lambda i: (0, 0)),
        ],
        out_specs=pl.BlockSpec((1, 1, BN), lambda i: (i, 0, 0)),
        out_shape=jax.ShapeDtypeStruct((N // BN, 1, BN), jnp.int32),
        scratch_shapes=[pltpu.VMEM((K, 1), jnp.float32)],
    )(x, centroids)
    return out.reshape(N)
